# Initial kernel scaffold; baseline (speedup 1.0000x reference)
import jax
import jax.numpy as jnp
from jax.experimental import pallas as pl


def kernel(q_sub, q_rel, hidden, edges, n_node, edge_head_rc_repr, edge_tail_rc_repr, query_head_rc_repr, rela_embed, time_pe, Ws, Wr, W1f, b1f, W2f, b2f, Wqr, bqr, w_alpha, Wg, bg, Wt, bt, Wh):
    raise NotImplementedError("write your pallas kernel here")


# speedup vs baseline: 3.2533x; 3.2533x over previous
import functools

import jax
import jax.numpy as jnp
from jax import lax
from jax.experimental import pallas as pl
from jax.experimental.pallas import tpu as pltpu
from jax.experimental.pallas import tpu_sc as plsc

E = 160000
D = 128
TD = 32
N = 10000
NQP = 10240
BLK = 2000
GB = 128
NBLK = E // GB
NC, NS = 2, 16
NW = NC * NS
NP = 10240
NPT = NP // NS

_sc_mesh = plsc.VectorSubcoreMesh(
    core_axis_name="c", subcore_axis_name="s", num_cores=NC, num_subcores=NS)


def _share(total, workers, w):
    return total // workers + jnp.where(w < total % workers, 1, 0)


def _qr_body(qrel_h, rela_h, qrt_o, idx_v, rows_v):
    cid = lax.axis_index("c")
    sid = lax.axis_index("s")
    wid = sid * NC + cid

    def body(i, carry):
        base = (wid + i * NW) * GB
        pltpu.sync_copy(qrel_h.at[pl.ds(base, GB)], idx_v)
        pltpu.sync_copy(rela_h.at[idx_v], rows_v)
        pltpu.sync_copy(rows_v, qrt_o.at[pl.ds(base, GB)])
        return carry

    lax.fori_loop(0, _share(NQP // GB, NW, wid), body, 0)


def _qr_stage(q_rel_pad, rela_embed):
    return pl.kernel(
        _qr_body,
        out_type=jax.ShapeDtypeStruct((NQP, D), jnp.float32),
        mesh=_sc_mesh,
        scratch_types=[pltpu.VMEM((GB,), jnp.int32),
                       pltpu.VMEM((GB, D), jnp.float32)],
    )(q_rel_pad, rela_embed)


def _gather_body(sub_h, e2_h, e6_h, ridx_h, hidden_h, rela_h, tpe_h, qrt_h,
                 hs_o, rel_o, tim_o, hqr_o,
                 sub_v, e2_v, e6_v, ridx_v,
                 hs_v, rel_v, tim_v, hqr_v, s0, s1, s2, s3):
    cid = lax.axis_index("c")
    sid = lax.axis_index("s")
    wid = sid * NC + cid

    def body(i, carry):
        base = (wid + i * NW) * GB
        pltpu.sync_copy(sub_h.at[pl.ds(base, GB)], sub_v)
        pltpu.sync_copy(e2_h.at[pl.ds(base, GB)], e2_v)
        pltpu.sync_copy(e6_h.at[pl.ds(base, GB)], e6_v)
        pltpu.sync_copy(ridx_h.at[pl.ds(base, GB)], ridx_v)
        c0 = pltpu.async_copy(hidden_h.at[sub_v], hs_v, s0)
        c1 = pltpu.async_copy(rela_h.at[e2_v], rel_v, s1)
        c2 = pltpu.async_copy(tpe_h.at[e6_v], tim_v, s2)
        c3 = pltpu.async_copy(qrt_h.at[ridx_v], hqr_v, s3)
        c0.wait()
        pltpu.sync_copy(hs_v, hs_o.at[pl.ds(base, GB)])
        c1.wait()
        pltpu.sync_copy(rel_v, rel_o.at[pl.ds(base, GB)])
        c2.wait()
        pltpu.sync_copy(tim_v, tim_o.at[pl.ds(base, GB)])
        c3.wait()
        pltpu.sync_copy(hqr_v, hqr_o.at[pl.ds(base, GB)])
        return carry

    lax.fori_loop(0, _share(NBLK, NW, wid), body, 0)


def _gather_stage(sub, e2, e6, r_idx, hidden, rela_embed, time_pe_pad, qr_table):
    f32 = jnp.float32
    i32 = jnp.int32
    return pl.kernel(
        _gather_body,
        out_type=[jax.ShapeDtypeStruct((E, D), f32),
                  jax.ShapeDtypeStruct((E, D), f32),
                  jax.ShapeDtypeStruct((E, D), f32),
                  jax.ShapeDtypeStruct((E, D), f32)],
        mesh=_sc_mesh,
        scratch_types=[pltpu.VMEM((GB,), i32), pltpu.VMEM((GB,), i32),
                       pltpu.VMEM((GB,), i32), pltpu.VMEM((GB,), i32),
                       pltpu.VMEM((GB, D), f32), pltpu.VMEM((GB, D), f32),
                       pltpu.VMEM((GB, D), f32), pltpu.VMEM((GB, D), f32),
                       pltpu.SemaphoreType.DMA, pltpu.SemaphoreType.DMA,
                       pltpu.SemaphoreType.DMA, pltpu.SemaphoreType.DMA],
    )(sub, e2, e6, r_idx, hidden, rela_embed, time_pe_pad, qr_table)


def _dense_body(rel_r, tim_r, hs_r, hqr_r, head_r, tail_r,
                W1a_r, W1b_r, b1_r, W2_r, b2_r,
                Wg1_r, Wg2_r, Wg3_r, Wg4_r, Wg5_r, bg_r,
                Wt1_r, Wt2_r, bt_r,
                Ws_r, Wr_r, Wq1_r, Wq2_r, Wq3_r, bqr_r, wa_r,
                up_r):
    lr = lambda x: jnp.where(x > 0, x, 0.01 * x)
    dot = lambda a, b: jnp.dot(a, b, preferred_element_type=jnp.float32)
    rel = rel_r[...]
    hs = hs_r[...]
    hqr = hqr_r[...]
    head = head_r[...]
    tail = tail_r[...]
    h1 = lr(dot(rel, W1a_r[...]) + dot(tim_r[...], W1b_r[...]) + b1_r[...])
    h2 = lr(dot(h1, W2_r[...]) + b2_r[...])
    hr = h2 + rel
    gin = (dot(hr, Wg1_r[...])
           + 0.25 * (dot(hqr, Wg2_r[...]) + dot(head, Wg3_r[...])
                     + dot(tail, Wg4_r[...]))
           + dot(hs, Wg5_r[...]) + bg_r[...])
    gates = jax.nn.sigmoid(gin)
    update = gates[:, :D]
    reset = gates[:, D:]
    cand = jnp.tanh(dot(hr, Wt1_r[...]) + dot(reset * hs, Wt2_r[...]) + bt_r[...])
    message = (1.0 - update) * hs + update * cand
    att = lr(dot(hs, Ws_r[...]) + dot(hr, Wr_r[...]) + dot(hqr, Wq1_r[...])
             + dot(head, Wq2_r[...]) + dot(tail, Wq3_r[...]) + bqr_r[...])
    alpha = dot(att, wa_r[...])
    up_r[...] = jax.nn.sigmoid(alpha) * message


def _dense_stage(rel, tim, hs, hqr, head, tail, weights):
    nblk = E // BLK
    row_spec = lambda w: pl.BlockSpec((BLK, w), lambda i: (i, 0))
    full = lambda a: pl.BlockSpec(a.shape, lambda i: (0,) * a.ndim)
    return pl.pallas_call(
        _dense_body,
        grid=(nblk,),
        in_specs=[row_spec(D), row_spec(D), row_spec(D), row_spec(D),
                  row_spec(D), row_spec(D)] + [full(w) for w in weights],
        out_specs=row_spec(D),
        out_shape=jax.ShapeDtypeStruct((E, D), jnp.float32),
    )(rel, tim, hs, hqr, head, tail, *weights)


def _scatter_body(obj_h, up_h, ones_h, zu_h,
                  pu_o, pd_o,
                  obj_v, up_v, acc):
    cid = lax.axis_index("c")
    sid = lax.axis_index("s")
    r0 = sid * NPT
    pltpu.sync_copy(zu_h.at[pl.ds(r0, NPT)], acc.at[pl.ds(r0, NPT)])
    plsc.subcore_barrier()

    @pl.when(cid == 0)
    def _up_core():
        def body(i, carry):
            base = (sid + i * NS) * GB
            pltpu.sync_copy(obj_h.at[pl.ds(base, GB)], obj_v)
            pltpu.sync_copy(up_h.at[pl.ds(base, GB)], up_v)
            pltpu.sync_copy(up_v, acc.at[obj_v], add=True)
            return carry

        lax.fori_loop(0, _share(NBLK, NS, sid), body, 0)

    @pl.when(cid == 1)
    def _deg_core():
        pltpu.sync_copy(ones_h, up_v)

        def body(i, carry):
            base = (sid + i * NS) * GB
            pltpu.sync_copy(obj_h.at[pl.ds(base, GB)], obj_v)
            pltpu.sync_copy(up_v, acc.at[obj_v], add=True)
            return carry

        lax.fori_loop(0, _share(NBLK, NS, sid), body, 0)

    plsc.subcore_barrier()

    @pl.when(cid == 0)
    def _out_up():
        pltpu.sync_copy(acc.at[pl.ds(r0, NPT)], pu_o.at[pl.ds(r0, NPT)])

    @pl.when(cid == 1)
    def _out_deg():
        pltpu.sync_copy(acc.at[pl.ds(r0, NPT)], pd_o.at[pl.ds(r0, NPT)])


def _scatter_stage(obj, up):
    f32 = jnp.float32
    ones = jnp.ones((GB, D), f32)
    zu = jnp.zeros((NP, D), f32)
    return pl.kernel(
        _scatter_body,
        out_type=[jax.ShapeDtypeStruct((NP, D), f32),
                  jax.ShapeDtypeStruct((NP, D), f32)],
        mesh=_sc_mesh,
        scratch_types=[pltpu.VMEM((GB,), jnp.int32),
                       pltpu.VMEM((GB, D), f32),
                       pltpu.VMEM_SHARED((NP, D), f32)],
    )(obj, up, ones, zu)


def _final_body(pu_r, pd_r, Wh_r, out_r):
    deg = pd_r[:, 0:1]
    agg = pu_r[...] / jnp.sqrt(deg + 0.0001)
    out_r[...] = jnp.dot(agg, Wh_r[...], preferred_element_type=jnp.float32)


def _final_stage(part_up, part_deg, Wh):
    rb = 2000
    return pl.pallas_call(
        _final_body,
        grid=(N // rb,),
        in_specs=[pl.BlockSpec((rb, D), lambda i: (i, 0)),
                  pl.BlockSpec((rb, D), lambda i: (i, 0)),
                  pl.BlockSpec(Wh.shape, lambda i: (0, 0))],
        out_specs=pl.BlockSpec((rb, D), lambda i: (i, 0)),
        out_shape=jax.ShapeDtypeStruct((N, D), jnp.float32),
    )(part_up, part_deg, Wh)


def kernel(q_sub, q_rel, hidden, edges, n_node, edge_head_rc_repr,
           edge_tail_rc_repr, query_head_rc_repr, rela_embed, time_pe,
           Ws, Wr, W1f, b1f, W2f, b2f, Wqr, bqr, w_alpha, Wg, bg,
           Wt, bt, Wh):
    sub = edges[:, 4]
    obj = edges[:, 5]
    r_idx = edges[:, 0]
    e2 = edges[:, 2]
    e6 = edges[:, 6]
    obj = obj + (jnp.asarray(n_node, dtype=obj.dtype) - N)

    q_rel_pad = jnp.pad(q_rel.astype(jnp.int32), (0, NQP - q_rel.shape[0]))
    qr_table = _qr_stage(q_rel_pad, rela_embed)

    time_pe_pad = jnp.pad(time_pe, ((0, 0), (0, D - TD)))
    hs, rel, tim, hqr = _gather_stage(sub, e2, e6, r_idx, hidden,
                                      rela_embed, time_pe_pad, qr_table)

    W1b_pad = jnp.pad(W1f[D:], ((0, D - TD), (0, 0)))
    weights = (
        W1f[:D], W1b_pad, b1f.reshape(1, D), W2f, b2f.reshape(1, D),
        Wg[:D], Wg[D:2 * D], Wg[2 * D:3 * D], Wg[3 * D:4 * D], Wg[4 * D:],
        bg.reshape(1, 2 * D),
        Wt[:D], Wt[D:], bt.reshape(1, D),
        Ws, Wr, Wqr[:D], Wqr[D:2 * D], Wqr[2 * D:], bqr.reshape(1, D),
        w_alpha,
    )
    up = _dense_stage(rel, tim, hs, hqr, edge_head_rc_repr,
                      edge_tail_rc_repr, weights)

    part_up, part_deg = _scatter_stage(obj.astype(jnp.int32), up)

    return _final_stage(part_up, part_deg, Wh)
